# Initial kernel scaffold; baseline (speedup 1.0000x reference)
import jax
import jax.numpy as jnp
from jax.experimental import pallas as pl


def kernel(hidden_states, gate_up_proj, down_proj, router_w, shared_gate_w, shared_up_w, shared_down_w):
    raise NotImplementedError("write your pallas kernel here")


# speedup vs baseline: 1.4671x; 1.4671x over previous
import functools

import jax
import jax.numpy as jnp
from jax.experimental import pallas as pl

NUM_EXPERTS = 8
HIDDEN = 1024
INTER = 2048
TCHUNK = 512


def _moe_body(hs_ref, rw_ref, gu_ref, dn_ref, sg_ref, su_ref, sd_ref, out_ref):
    e = pl.program_id(1)
    x32 = hs_ref[...]
    logits = jnp.dot(x32, rw_ref[...], preferred_element_type=jnp.float32)
    m = jnp.max(logits, axis=1, keepdims=True)
    s = jax.nn.sigmoid(m)
    iota_e = jax.lax.broadcasted_iota(jnp.int32, logits.shape, 1)
    idx = jnp.min(jnp.where(logits == m, iota_e, NUM_EXPERTS), axis=1,
                  keepdims=True)
    xb = ((x32 * s) * (idx == e).astype(jnp.float32)).astype(jnp.bfloat16)
    gu = jnp.dot(xb, gu_ref[0], preferred_element_type=jnp.float32)
    gate = gu[:, :INTER]
    up = gu[:, INTER:]
    gated = (up * (gate * jax.nn.sigmoid(gate))).astype(jnp.bfloat16)
    contrib = jnp.dot(gated, dn_ref[0], preferred_element_type=jnp.float32)

    @pl.when(e == 0)
    def _():
        xs = x32.astype(jnp.bfloat16)
        g = jnp.dot(xs, sg_ref[...], preferred_element_type=jnp.float32)
        u = jnp.dot(xs, su_ref[...], preferred_element_type=jnp.float32)
        act = (u * (g * jax.nn.sigmoid(g))).astype(jnp.bfloat16)
        sh = jnp.dot(act, sd_ref[...], preferred_element_type=jnp.float32)
        out_ref[...] = sh + contrib

    @pl.when(e != 0)
    def _():
        out_ref[...] = out_ref[...] + contrib


def kernel(hidden_states, gate_up_proj, down_proj, router_w, shared_gate_w,
           shared_up_w, shared_down_w):
    B, S, H = hidden_states.shape
    T = B * S
    hs = hidden_states.reshape(T, H)
    rw_t = router_w.T.astype(jnp.float32)
    gu_bf = gate_up_proj.astype(jnp.bfloat16)
    dn_bf = down_proj.astype(jnp.bfloat16)
    sg_t = shared_gate_w.T.astype(jnp.bfloat16)
    su_t = shared_up_w.T.astype(jnp.bfloat16)
    sd_t = shared_down_w.T.astype(jnp.bfloat16)

    n_chunks = T // TCHUNK
    grid = (n_chunks, NUM_EXPERTS)
    out = pl.pallas_call(
        _moe_body,
        grid=grid,
        in_specs=[
            pl.BlockSpec((TCHUNK, H), lambda t, e: (t, 0)),
            pl.BlockSpec((H, NUM_EXPERTS), lambda t, e: (0, 0)),
            pl.BlockSpec((1, H, 2 * INTER), lambda t, e: (e, 0, 0)),
            pl.BlockSpec((1, INTER, H), lambda t, e: (e, 0, 0)),
            pl.BlockSpec((H, INTER), lambda t, e: (0, 0)),
            pl.BlockSpec((H, INTER), lambda t, e: (0, 0)),
            pl.BlockSpec((INTER, H), lambda t, e: (0, 0)),
        ],
        out_specs=pl.BlockSpec((TCHUNK, H), lambda t, e: (t, 0)),
        out_shape=jax.ShapeDtypeStruct((T, H), jnp.float32),
    )(hs, rw_t, gu_bf, dn_bf, sg_t, su_t, sd_t)
    return out


# speedup vs baseline: 2.8829x; 1.9650x over previous
import functools

import jax
import jax.numpy as jnp
from jax import lax
from jax.experimental import pallas as pl
from jax.experimental.pallas import tpu as pltpu
from jax.experimental.pallas import tpu_sc as plsc

NUM_EXPERTS = 8
HIDDEN = 1024
INTER = 2048
T = 4096
TCHUNK = 512
RCHUNKS = T // TCHUNK
BLK = 256
P = T + NUM_EXPERTS * BLK
NB = P // BLK
NC, NS = 2, 16
NW = NC * NS
TPW = T // NW
HALF = TPW // 2


def _router_body(hs_ref, rw_ref, xs_ref, idx_ref, rank_ref, counts_ref,
                 cnt_scr):
    t = pl.program_id(0)

    @pl.when(t == 0)
    def _():
        cnt_scr[...] = jnp.zeros((1, 128), jnp.float32)

    x32 = hs_ref[...]
    logits = jnp.dot(x32, rw_ref[...], preferred_element_type=jnp.float32)
    m = jnp.max(logits, axis=1, keepdims=True)
    s = jax.nn.sigmoid(m)
    iota_e = lax.broadcasted_iota(jnp.int32, logits.shape, 1)
    idx = jnp.min(jnp.where(logits == m, iota_e, NUM_EXPERTS), axis=1,
                  keepdims=True)
    onehot = (iota_e == idx).astype(jnp.float32)
    ri = lax.broadcasted_iota(jnp.int32, (TCHUNK, TCHUNK), 0)
    ci = lax.broadcasted_iota(jnp.int32, (TCHUNK, TCHUNK), 1)
    tril = (ci < ri).astype(jnp.float32)
    prev = jnp.dot(tril, onehot, preferred_element_type=jnp.float32)
    rank_local = jnp.sum(prev * onehot, axis=1, keepdims=True)
    cnt = cnt_scr[:, :NUM_EXPERTS]
    carry = jnp.sum(jnp.broadcast_to(cnt, onehot.shape) * onehot, axis=1,
                    keepdims=True)
    idx_ref[...] = idx
    rank_ref[...] = (rank_local + carry).astype(jnp.int32)
    xs_ref[...] = x32 * s
    cnt_scr[:, :NUM_EXPERTS] = cnt + jnp.sum(onehot, axis=0, keepdims=True)

    @pl.when(t == RCHUNKS - 1)
    def _():
        counts_ref[...] = cnt_scr[...]


def _run_router(hs, rw_t):
    return pl.pallas_call(
        _router_body,
        grid=(RCHUNKS,),
        in_specs=[
            pl.BlockSpec((TCHUNK, HIDDEN), lambda t: (t, 0)),
            pl.BlockSpec((HIDDEN, NUM_EXPERTS), lambda t: (0, 0)),
        ],
        out_specs=[
            pl.BlockSpec((TCHUNK, HIDDEN), lambda t: (t, 0)),
            pl.BlockSpec((TCHUNK, 1), lambda t: (t, 0)),
            pl.BlockSpec((TCHUNK, 1), lambda t: (t, 0)),
            pl.BlockSpec((1, 128), lambda t: (0, 0)),
        ],
        out_shape=[
            jax.ShapeDtypeStruct((T, HIDDEN), jnp.float32),
            jax.ShapeDtypeStruct((T, 1), jnp.int32),
            jax.ShapeDtypeStruct((T, 1), jnp.int32),
            jax.ShapeDtypeStruct((1, 128), jnp.float32),
        ],
        scratch_shapes=[pltpu.VMEM((1, 128), jnp.float32)],
    )(hs, rw_t)


def _slot_body(idx_ref, rank_ref, counts_ref, slot_ref):
    idx = idx_ref[...]
    cnt = counts_ref[:, :NUM_EXPERTS]
    g_pad = jnp.ceil(cnt / BLK) * BLK
    r8 = lax.broadcasted_iota(jnp.int32, (NUM_EXPERTS, NUM_EXPERTS), 0)
    c8 = lax.broadcasted_iota(jnp.int32, (NUM_EXPERTS, NUM_EXPERTS), 1)
    strict_upper = (r8 < c8).astype(jnp.float32)
    base = jnp.dot(g_pad, strict_upper, preferred_element_type=jnp.float32)
    iota_e = lax.broadcasted_iota(jnp.int32, (T, NUM_EXPERTS), 1)
    oh = (iota_e == idx).astype(jnp.float32)
    basetok = jnp.sum(jnp.broadcast_to(base, oh.shape) * oh, axis=1,
                      keepdims=True)
    slot_ref[...] = rank_ref[...] + basetok.astype(jnp.int32)


def _run_slot(idx2, rank2, counts_row):
    return pl.pallas_call(
        _slot_body,
        grid=(1,),
        in_specs=[
            pl.BlockSpec((T, 1), lambda i: (0, 0)),
            pl.BlockSpec((T, 1), lambda i: (0, 0)),
            pl.BlockSpec((1, 128), lambda i: (0, 0)),
        ],
        out_specs=pl.BlockSpec((T, 1), lambda i: (0, 0)),
        out_shape=jax.ShapeDtypeStruct((T, 1), jnp.int32),
    )(idx2, rank2, counts_row)


@functools.lru_cache(maxsize=None)
def _sc_kernels():
    mesh = plsc.VectorSubcoreMesh(core_axis_name="c", subcore_axis_name="s",
                                  num_cores=NC, num_subcores=NS)

    @functools.partial(
        pl.kernel,
        out_type=jax.ShapeDtypeStruct((P, HIDDEN), jnp.float32),
        mesh=mesh,
        scratch_types=[
            pltpu.VMEM((HALF,), jnp.int32),
            pltpu.VMEM((HALF,), jnp.int32),
            pltpu.VMEM((HALF, HIDDEN), jnp.float32),
            pltpu.SemaphoreType.DMA,
        ],
    )
    def _scatter_kernel(xs_hbm, slot_hbm, xsorted_hbm, slot_lo, slot_hi,
                        rows_v, sem):
        wid = lax.axis_index("s") * NC + lax.axis_index("c")
        t0 = wid * TPW
        pltpu.sync_copy(slot_hbm.at[pl.ds(t0, HALF)], slot_lo)
        pltpu.sync_copy(slot_hbm.at[pl.ds(t0 + HALF, HALF)], slot_hi)
        for h, sl in ((0, slot_lo), (1, slot_hi)):
            pltpu.sync_copy(xs_hbm.at[pl.ds(t0 + h * HALF, HALF)], rows_v)
            pltpu.async_copy(rows_v, xsorted_hbm.at[sl], sem).wait()

    @functools.partial(
        pl.kernel,
        out_type=jax.ShapeDtypeStruct((T, HIDDEN), jnp.float32),
        mesh=mesh,
        scratch_types=[
            pltpu.VMEM((HALF,), jnp.int32),
            pltpu.VMEM((HALF,), jnp.int32),
            pltpu.VMEM((HALF, HIDDEN), jnp.float32),
            pltpu.SemaphoreType.DMA,
        ],
    )
    def _gather_kernel(ys_hbm, slot_hbm, out_hbm, slot_lo, slot_hi, rows_v,
                       sem):
        wid = lax.axis_index("s") * NC + lax.axis_index("c")
        t0 = wid * TPW
        pltpu.sync_copy(slot_hbm.at[pl.ds(t0, HALF)], slot_lo)
        pltpu.sync_copy(slot_hbm.at[pl.ds(t0 + HALF, HALF)], slot_hi)
        for h, sl in ((0, slot_lo), (1, slot_hi)):
            pltpu.async_copy(ys_hbm.at[sl], rows_v, sem).wait()
            pltpu.sync_copy(rows_v, out_hbm.at[pl.ds(t0 + h * HALF, HALF)])

    return _scatter_kernel, _gather_kernel


def _ffn_body(be_ref, xs_ref, gu_ref, dn_ref, out_ref):
    x = xs_ref[...].astype(jnp.bfloat16)
    gu = jnp.dot(x, gu_ref[0], preferred_element_type=jnp.float32)
    gate = gu[:, :INTER]
    up = gu[:, INTER:]
    gated = (up * (gate * jax.nn.sigmoid(gate))).astype(jnp.bfloat16)
    out_ref[...] = jnp.dot(gated, dn_ref[0],
                           preferred_element_type=jnp.float32)


def _run_ffn(bexp, xsorted, gu_bf, dn_bf):
    grid_spec = pltpu.PrefetchScalarGridSpec(
        num_scalar_prefetch=1,
        grid=(NB,),
        in_specs=[
            pl.BlockSpec((BLK, HIDDEN), lambda j, be: (j, 0)),
            pl.BlockSpec((1, HIDDEN, 2 * INTER), lambda j, be: (be[j], 0, 0)),
            pl.BlockSpec((1, INTER, HIDDEN), lambda j, be: (be[j], 0, 0)),
        ],
        out_specs=pl.BlockSpec((BLK, HIDDEN), lambda j, be: (j, 0)),
    )
    return pl.pallas_call(
        _ffn_body,
        grid_spec=grid_spec,
        out_shape=jax.ShapeDtypeStruct((P, HIDDEN), jnp.float32),
    )(bexp, xsorted, gu_bf, dn_bf)


def _shared_body(hs_ref, yt_ref, sg_ref, su_ref, sd_ref, out_ref):
    x = hs_ref[...].astype(jnp.bfloat16)
    g = jnp.dot(x, sg_ref[...], preferred_element_type=jnp.float32)
    u = jnp.dot(x, su_ref[...], preferred_element_type=jnp.float32)
    act = (u * (g * jax.nn.sigmoid(g))).astype(jnp.bfloat16)
    sh = jnp.dot(act, sd_ref[...], preferred_element_type=jnp.float32)
    out_ref[...] = sh + yt_ref[...]


def _run_shared(hs, ys_tok, sg_t, su_t, sd_t):
    return pl.pallas_call(
        _shared_body,
        grid=(RCHUNKS,),
        in_specs=[
            pl.BlockSpec((TCHUNK, HIDDEN), lambda t: (t, 0)),
            pl.BlockSpec((TCHUNK, HIDDEN), lambda t: (t, 0)),
            pl.BlockSpec((HIDDEN, INTER), lambda t: (0, 0)),
            pl.BlockSpec((HIDDEN, INTER), lambda t: (0, 0)),
            pl.BlockSpec((INTER, HIDDEN), lambda t: (0, 0)),
        ],
        out_specs=pl.BlockSpec((TCHUNK, HIDDEN), lambda t: (t, 0)),
        out_shape=jax.ShapeDtypeStruct((T, HIDDEN), jnp.float32),
    )(hs, ys_tok, sg_t, su_t, sd_t)


def kernel(hidden_states, gate_up_proj, down_proj, router_w, shared_gate_w,
           shared_up_w, shared_down_w):
    B, S, H = hidden_states.shape
    hs = hidden_states.reshape(B * S, H)
    rw_t = router_w.T.astype(jnp.float32)
    gu_bf = gate_up_proj.astype(jnp.bfloat16)
    dn_bf = down_proj.astype(jnp.bfloat16)
    sg_t = shared_gate_w.T.astype(jnp.bfloat16)
    su_t = shared_up_w.T.astype(jnp.bfloat16)
    sd_t = shared_down_w.T.astype(jnp.bfloat16)

    xs, idx2, rank2, counts_row = _run_router(hs, rw_t)
    slot = _run_slot(idx2, rank2, counts_row).reshape(T)

    counts = counts_row[0, :NUM_EXPERTS]
    g_pad = jnp.ceil(counts / BLK) * BLK
    ends = jnp.cumsum(g_pad)
    jblk = jnp.arange(NB, dtype=jnp.float32) * BLK
    bexp = jnp.minimum(
        jnp.sum((jblk[:, None] >= ends[None, :]).astype(jnp.int32), axis=1),
        NUM_EXPERTS - 1).astype(jnp.int32)

    scatter_k, gather_k = _sc_kernels()
    xsorted = scatter_k(xs, slot)
    ys = _run_ffn(bexp, xsorted, gu_bf, dn_bf)
    ys_tok = gather_k(ys, slot)
    return _run_shared(hs, ys_tok, sg_t, su_t, sd_t)
